# Initial kernel scaffold; baseline (speedup 1.0000x reference)
import jax
import jax.numpy as jnp
from jax.experimental import pallas as pl


def kernel(x, edge_index, batch, params):
    raise NotImplementedError("write your pallas kernel here")


# speedup vs baseline: 1.0249x; 1.0249x over previous
import functools
import math

import jax
import jax.numpy as jnp
import numpy as np
from jax.experimental import pallas as pl
from jax.experimental.pallas import tpu as pltpu

N_NODES = 10000
N_EDGES = 320000
HID = 128
AVG_LOG = float(np.log(33.0))
BLK = 512


def _dense_body(with_last, s_ref, q_ref, mx_ref, mn_ref, deg_ref, h_ref,
                w_ref, cb_ref, g_ref, bb_ref, fw_ref, fb_ref,
                wih_ref, whh_ref, bih_ref, bhh_ref, lw_ref, lb_ref,
                h_out_ref, y_out_ref):
    deg = deg_ref[:]
    degc = jnp.maximum(deg, 1.0)
    inv = 1.0 / degc
    mean = s_ref[:] * inv
    var = jnp.maximum(q_ref[:] * inv - mean * mean, 0.0)
    std = jnp.sqrt(var + 1e-5)
    has = deg > 0.0
    mx = jnp.where(has, mx_ref[:], 0.0)
    mn = jnp.where(has, mn_ref[:], 0.0)
    agg = jnp.concatenate([mean, mn, mx, std], axis=1)
    p = jnp.dot(agg, w_ref[:], preferred_element_type=jnp.float32)
    logd = jnp.log(degc + 1.0)
    sc1 = logd * (1.0 / AVG_LOG)
    sc2 = AVG_LOG / logd
    x = p[:, :128] + sc1 * p[:, 128:256] + sc2 * p[:, 256:384] + cb_ref[:]
    x = x * (g_ref[:] * (1.0 / math.sqrt(1.0 + 1e-5))) + bb_ref[:]
    x = jnp.dot(x, fw_ref[:], preferred_element_type=jnp.float32) + fb_ref[:]
    x = jnp.maximum(x, 0.0)
    h = h_ref[:]
    gi = jnp.dot(x, wih_ref[:], preferred_element_type=jnp.float32) + bih_ref[:]
    gh = jnp.dot(h, whh_ref[:], preferred_element_type=jnp.float32) + bhh_ref[:]
    r = jax.nn.sigmoid(gi[:, :128] + gh[:, :128])
    z = jax.nn.sigmoid(gi[:, 128:256] + gh[:, 128:256])
    ng = jnp.tanh(gi[:, 256:] + r * gh[:, 256:])
    hn = (1.0 - z) * ng + z * h
    h_out_ref[:] = hn
    if with_last:
        y_out_ref[:] = (jnp.dot(hn, lw_ref[:], preferred_element_type=jnp.float32)
                        + lb_ref[:])
    else:
        y_out_ref[:] = hn


def _row_spec():
    return pl.BlockSpec((BLK, HID), lambda i: (i, 0))


def _full_spec(shape):
    nd = len(shape)
    return pl.BlockSpec(shape, lambda i, _nd=nd: (0,) * nd)


def _dense_layer(with_last, s, q, mx, mn, deg, h, w_all, cb, g, bb, fw, fb,
                 wih, whh, bih, bhh, lw, lb):
    grid = (pl.cdiv(N_NODES, BLK),)
    in_specs = [
        _row_spec(), _row_spec(), _row_spec(), _row_spec(),
        pl.BlockSpec((BLK, 1), lambda i: (i, 0)),
        _row_spec(),
        _full_spec((4 * HID, 3 * HID)), _full_spec((HID,)),
        _full_spec((HID,)), _full_spec((HID,)),
        _full_spec((HID, HID)), _full_spec((HID,)),
        _full_spec((HID, 3 * HID)), _full_spec((HID, 3 * HID)),
        _full_spec((3 * HID,)), _full_spec((3 * HID,)),
        _full_spec((HID, HID)), _full_spec((HID,)),
    ]
    out_specs = [_row_spec(), _row_spec()]
    out_shape = [jax.ShapeDtypeStruct((N_NODES, HID), jnp.float32),
                 jax.ShapeDtypeStruct((N_NODES, HID), jnp.float32)]
    fn = pl.pallas_call(
        functools.partial(_dense_body, with_last),
        grid=grid, in_specs=in_specs, out_specs=out_specs,
        out_shape=out_shape)
    return fn(s, q, mx, mn, deg, h, w_all, cb, g, bb, fw, fb,
              wih, whh, bih, bhh, lw, lb)


def _aggregate(x, src, dst):
    msg = x[src]
    s = jax.ops.segment_sum(msg, dst, num_segments=N_NODES)
    q = jax.ops.segment_sum(msg * msg, dst, num_segments=N_NODES)
    mx = jax.ops.segment_max(msg, dst, num_segments=N_NODES)
    mn = jax.ops.segment_min(msg, dst, num_segments=N_NODES)
    return s, q, mx, mn


def kernel(x, edge_index, batch, params):
    src = edge_index[0]
    dst = edge_index[1]
    deg = jax.ops.segment_sum(jnp.ones((N_EDGES,), jnp.float32), dst,
                              num_segments=N_NODES).reshape(N_NODES, 1)

    w_alls, cbs, gs, bbs, fws, fbs = [], [], [], [], [], []
    for i in range(3):
        w = params['conv%d_w' % i]
        wt = w.T
        w_all = jnp.concatenate([wt[:512], wt[512:1024], wt[1024:]], axis=1)
        w_alls.append(w_all)
        cbs.append(params['conv%d_b' % i])
        gs.append(params['bn%d_g' % i])
        bbs.append(params['bn%d_b' % i])
        fws.append(params['fc%d_w' % i].T)
        fbs.append(params['fc%d_b' % i])
    wih = params['gru_w_ih'].T
    whh = params['gru_w_hh'].T
    bih = params['gru_b_ih']
    bhh = params['gru_b_hh']
    lw = params['last_w'].T
    lb = params['last_b']

    h = jnp.zeros((N_NODES, HID), jnp.float32)
    cur = x
    y = None
    for i in range(3):
        s, q, mx, mn = _aggregate(cur, src, dst)
        with_last = (i == 2)
        h, y = _dense_layer(with_last, s, q, mx, mn, deg, h,
                            w_alls[i], cbs[i], gs[i], bbs[i], fws[i], fbs[i],
                            wih, whh, bih, bhh, lw, lb)
        cur = h
    return y
